# Initial kernel scaffold; baseline (speedup 1.0000x reference)
import functools

import jax
import jax.numpy as jnp
from jax import lax
from jax.experimental import pallas as pl
from jax.experimental.pallas import tpu as pltpu
from jax.experimental.pallas import tpu_sc as plsc

N = 10000
E = 320000
F = 128
C = 10
CP = 16
NC, NS = 2, 16
NW = NC * NS
BK = 128
NB = 80
EPT = NB * BK
E_PAD = NW * EPT
R = 10240
RT = R // NS
TRASH = N + 16
BR = 1000
GRID = N // BR


def _mesh():
    return plsc.VectorSubcoreMesh(
        core_axis_name="c", subcore_axis_name="s", num_cores=NC, num_subcores=NS)


def _sc_deg():
    @functools.partial(
        pl.kernel,
        out_type=jax.ShapeDtypeStruct((NC, R, CP), jnp.float32),
        mesh=_mesh(),
        scratch_types=[
            pltpu.VMEM((NB, BK), jnp.int32),
            pltpu.VMEM((BK, CP), jnp.float32),
            pltpu.VMEM_SHARED((R, CP), jnp.float32),
        ],
        name="gcn_deg",
    )
    def deg(dst_hbm, ones_hbm, zeros_hbm, degp_hbm, dst_v, ones_v, acc_sh):
        cid = lax.axis_index("c")
        sid = lax.axis_index("s")
        w = cid * NS + sid
        pltpu.sync_copy(dst_hbm.at[w], dst_v)
        pltpu.sync_copy(ones_hbm, ones_v)
        pltpu.sync_copy(zeros_hbm.at[pl.ds(sid * RT, RT)],
                        acc_sh.at[pl.ds(sid * RT, RT)])
        plsc.subcore_barrier()

        def step(j, carry):
            pltpu.sync_copy(ones_v, acc_sh.at[dst_v.at[j]], add=True)
            return carry

        lax.fori_loop(0, NB, step, 0)
        plsc.subcore_barrier()
        pltpu.sync_copy(acc_sh.at[pl.ds(sid * RT, RT)],
                        degp_hbm.at[cid, pl.ds(sid * RT, RT)])

    return deg


def _sc_prop(D):
    @functools.partial(
        pl.kernel,
        out_type=jax.ShapeDtypeStruct((NC, R, D), jnp.float32),
        mesh=_mesh(),
        scratch_types=[
            pltpu.VMEM((NB, BK), jnp.int32),
            pltpu.VMEM((NB, BK), jnp.int32),
            pltpu.VMEM((BK, D), jnp.float32),
            pltpu.SemaphoreType.DMA,
            pltpu.VMEM_SHARED((R, D), jnp.float32),
        ],
        name=f"gcn_prop_d{D}",
    )
    def prop(y_hbm, src_hbm, dst_hbm, zeros_hbm, p_hbm, src_v, dst_v, buf, sem,
             acc_sh):
        cid = lax.axis_index("c")
        sid = lax.axis_index("s")
        w = cid * NS + sid
        pltpu.sync_copy(src_hbm.at[w], src_v)
        pltpu.sync_copy(dst_hbm.at[w], dst_v)
        pltpu.sync_copy(zeros_hbm.at[pl.ds(sid * RT, RT)],
                        acc_sh.at[pl.ds(sid * RT, RT)])
        plsc.subcore_barrier()

        def step(j, carry):
            pltpu.async_copy(y_hbm.at[src_v.at[j]], buf, sem).wait()
            pltpu.sync_copy(buf, acc_sh.at[dst_v.at[j]], add=True)
            return carry

        lax.fori_loop(0, NB, step, 0)
        plsc.subcore_barrier()
        pltpu.sync_copy(acc_sh.at[pl.ds(sid * RT, RT)],
                        p_hbm.at[cid, pl.ds(sid * RT, RT)])

    return prop


def _dinv(degp_ref):
    deg = degp_ref[0, :, 0:1] + degp_ref[1, :, 0:1] + 1.0
    return lax.rsqrt(deg)


def _tc_scale_in(x, W1, degp):
    def body(x_ref, w_ref, degp_ref, y_ref):
        dinv = _dinv(degp_ref)
        xw = jnp.dot(x_ref[...], w_ref[...], preferred_element_type=jnp.float32)
        y_ref[...] = xw * dinv

    return pl.pallas_call(
        body,
        grid=(GRID,),
        in_specs=[
            pl.BlockSpec((BR, F), lambda i: (i, 0)),
            pl.BlockSpec((F, F), lambda i: (0, 0)),
            pl.BlockSpec((NC, BR, CP), lambda i: (0, i, 0)),
        ],
        out_specs=pl.BlockSpec((BR, F), lambda i: (i, 0)),
        out_shape=jax.ShapeDtypeStruct((N, F), jnp.float32),
    )(x, W1, degp)


def _tc_mid(p, degp, y1, b1r, W2p):
    def body(p_ref, degp_ref, y1_ref, b1_ref, w2_ref, y2_ref):
        dinv = _dinv(degp_ref)
        acc = p_ref[0] + p_ref[1] + y1_ref[...]
        h = jnp.maximum(acc * dinv + b1_ref[...], 0.0)
        hw = jnp.dot(h, w2_ref[...], preferred_element_type=jnp.float32)
        y2_ref[...] = hw * dinv

    return pl.pallas_call(
        body,
        grid=(GRID,),
        in_specs=[
            pl.BlockSpec((NC, BR, F), lambda i: (0, i, 0)),
            pl.BlockSpec((NC, BR, CP), lambda i: (0, i, 0)),
            pl.BlockSpec((BR, F), lambda i: (i, 0)),
            pl.BlockSpec((1, F), lambda i: (0, 0)),
            pl.BlockSpec((F, CP), lambda i: (0, 0)),
        ],
        out_specs=pl.BlockSpec((BR, CP), lambda i: (i, 0)),
        out_shape=jax.ShapeDtypeStruct((N, CP), jnp.float32),
    )(p, degp, y1, b1r, W2p)


def _tc_out(q, degp, y2, b2r):
    def body(q_ref, degp_ref, y2_ref, b2_ref, o_ref):
        dinv = _dinv(degp_ref)
        logits = (q_ref[0] + q_ref[1] + y2_ref[...]) * dinv + b2_ref[...]
        col = lax.broadcasted_iota(jnp.int32, (BR, CP), 1)
        logits = jnp.where(col < C, logits, -1e30)
        m = jnp.max(logits, axis=1, keepdims=True)
        s = jnp.sum(jnp.exp(logits - m), axis=1, keepdims=True)
        o_ref[...] = logits - m - jnp.log(s)

    return pl.pallas_call(
        body,
        grid=(GRID,),
        in_specs=[
            pl.BlockSpec((NC, BR, CP), lambda i: (0, i, 0)),
            pl.BlockSpec((NC, BR, CP), lambda i: (0, i, 0)),
            pl.BlockSpec((BR, CP), lambda i: (i, 0)),
            pl.BlockSpec((1, CP), lambda i: (0, 0)),
        ],
        out_specs=pl.BlockSpec((BR, CP), lambda i: (i, 0)),
        out_shape=jax.ShapeDtypeStruct((N, CP), jnp.float32),
    )(q, degp, y2, b2r)


def kernel(x, edge_index, W1, b1, W2, b2):
    src = edge_index[0].astype(jnp.int32)
    dst = edge_index[1].astype(jnp.int32)
    src_p = jnp.pad(src, (0, E_PAD - E)).reshape(NW, NB, BK)
    dst_p = jnp.pad(dst, (0, E_PAD - E), constant_values=TRASH).reshape(NW, NB, BK)
    zeros_f = jnp.zeros((R, F), jnp.float32)
    zeros_c = jnp.zeros((R, CP), jnp.float32)
    ones_rows = jnp.ones((BK, CP), jnp.float32)
    b1r = jnp.reshape(b1, (1, F))
    W2p = jnp.pad(W2, ((0, 0), (0, CP - C)))
    b2r = jnp.reshape(jnp.pad(b2, (0, CP - C)), (1, CP))

    degp = _sc_deg()(dst_p, ones_rows, zeros_c)
    y1 = _tc_scale_in(x, W1, degp)
    p = _sc_prop(F)(y1, src_p, dst_p, zeros_f)
    y2 = _tc_mid(p, degp, y1, b1r, W2p)
    q = _sc_prop(CP)(y2, src_p, dst_p, zeros_c)
    out16 = _tc_out(q, degp, y2, b2r)
    return out16[:, :C]


# speedup vs baseline: 11.7102x; 11.7102x over previous
import functools

import jax
import jax.numpy as jnp
from jax import lax
from jax.experimental import pallas as pl
from jax.experimental.pallas import tpu as pltpu
from jax.experimental.pallas import tpu_sc as plsc

N = 10000
E = 320000
F = 128
C = 10
CP = 16
NC, NS = 2, 16
NW = NC * NS
BK = 128
NB = 80
EPT = NB * BK
E_PAD = NW * EPT
R = 10240
RT = R // NS
TRASH = N + 16
BR = 1000
GRID = N // BR


def _mesh():
    return plsc.VectorSubcoreMesh(
        core_axis_name="c", subcore_axis_name="s", num_cores=NC, num_subcores=NS)


def _sc_deg():
    @functools.partial(
        pl.kernel,
        out_type=jax.ShapeDtypeStruct((NC, R, CP), jnp.float32),
        mesh=_mesh(),
        scratch_types=[
            pltpu.VMEM((NB, BK), jnp.int32),
            pltpu.VMEM((BK, CP), jnp.float32),
            pltpu.VMEM_SHARED((R, CP), jnp.float32),
        ],
        compiler_params=pltpu.CompilerParams(use_tc_tiling_on_sc=False),
        name="gcn_deg",
    )
    def deg(dst_hbm, ones_hbm, zeros_hbm, degp_hbm, dst_v, ones_v, acc_sh):
        cid = lax.axis_index("c")
        sid = lax.axis_index("s")
        w = cid * NS + sid
        pltpu.sync_copy(dst_hbm.at[w], dst_v)
        pltpu.sync_copy(ones_hbm, ones_v)
        pltpu.sync_copy(zeros_hbm.at[pl.ds(sid * RT, RT)],
                        acc_sh.at[pl.ds(sid * RT, RT)])
        plsc.subcore_barrier()

        def step(j, carry):
            pltpu.sync_copy(ones_v, acc_sh.at[dst_v.at[j]], add=True)
            return carry

        lax.fori_loop(0, NB, step, 0)
        plsc.subcore_barrier()
        pltpu.sync_copy(acc_sh.at[pl.ds(sid * RT, RT)],
                        degp_hbm.at[cid, pl.ds(sid * RT, RT)])

    return deg


def _sc_prop(D):
    @functools.partial(
        pl.kernel,
        out_type=jax.ShapeDtypeStruct((NC, R, D), jnp.float32),
        mesh=_mesh(),
        scratch_types=[
            pltpu.VMEM((NB, BK), jnp.int32),
            pltpu.VMEM((NB, BK), jnp.int32),
            pltpu.VMEM((BK, D), jnp.float32),
            pltpu.SemaphoreType.DMA,
            pltpu.VMEM_SHARED((R, D), jnp.float32),
        ],
        compiler_params=pltpu.CompilerParams(use_tc_tiling_on_sc=False),
        name=f"gcn_prop_d{D}",
    )
    def prop(y_hbm, src_hbm, dst_hbm, zeros_hbm, p_hbm, src_v, dst_v, buf, sem,
             acc_sh):
        cid = lax.axis_index("c")
        sid = lax.axis_index("s")
        w = cid * NS + sid
        pltpu.sync_copy(src_hbm.at[w], src_v)
        pltpu.sync_copy(dst_hbm.at[w], dst_v)
        pltpu.sync_copy(zeros_hbm.at[pl.ds(sid * RT, RT)],
                        acc_sh.at[pl.ds(sid * RT, RT)])
        plsc.subcore_barrier()

        def step(j, carry):
            pltpu.async_copy(y_hbm.at[src_v.at[j]], buf, sem).wait()
            pltpu.sync_copy(buf, acc_sh.at[dst_v.at[j]], add=True)
            return carry

        lax.fori_loop(0, NB, step, 0)
        plsc.subcore_barrier()
        pltpu.sync_copy(acc_sh.at[pl.ds(sid * RT, RT)],
                        p_hbm.at[cid, pl.ds(sid * RT, RT)])

    return prop


def _dinv(degp_ref):
    deg = degp_ref[0, :, 0:1] + degp_ref[1, :, 0:1] + 1.0
    return lax.rsqrt(deg)


def _tc_scale_in(x, W1, degp):
    def body(x_ref, w_ref, degp_ref, y_ref):
        dinv = _dinv(degp_ref)
        xw = jnp.dot(x_ref[...], w_ref[...], preferred_element_type=jnp.float32)
        y_ref[...] = xw * dinv

    return pl.pallas_call(
        body,
        grid=(GRID,),
        in_specs=[
            pl.BlockSpec((BR, F), lambda i: (i, 0)),
            pl.BlockSpec((F, F), lambda i: (0, 0)),
            pl.BlockSpec((NC, BR, CP), lambda i: (0, i, 0)),
        ],
        out_specs=pl.BlockSpec((BR, F), lambda i: (i, 0)),
        out_shape=jax.ShapeDtypeStruct((N, F), jnp.float32),
    )(x, W1, degp)


def _tc_mid(p, degp, y1, b1r, W2p):
    def body(p_ref, degp_ref, y1_ref, b1_ref, w2_ref, y2_ref):
        dinv = _dinv(degp_ref)
        acc = p_ref[0] + p_ref[1] + y1_ref[...]
        h = jnp.maximum(acc * dinv + b1_ref[...], 0.0)
        hw = jnp.dot(h, w2_ref[...], preferred_element_type=jnp.float32)
        y2_ref[...] = hw * dinv

    return pl.pallas_call(
        body,
        grid=(GRID,),
        in_specs=[
            pl.BlockSpec((NC, BR, F), lambda i: (0, i, 0)),
            pl.BlockSpec((NC, BR, CP), lambda i: (0, i, 0)),
            pl.BlockSpec((BR, F), lambda i: (i, 0)),
            pl.BlockSpec((1, F), lambda i: (0, 0)),
            pl.BlockSpec((F, CP), lambda i: (0, 0)),
        ],
        out_specs=pl.BlockSpec((BR, CP), lambda i: (i, 0)),
        out_shape=jax.ShapeDtypeStruct((N, CP), jnp.float32),
    )(p, degp, y1, b1r, W2p)


def _tc_out(q, degp, y2, b2r):
    def body(q_ref, degp_ref, y2_ref, b2_ref, o_ref):
        dinv = _dinv(degp_ref)
        logits = (q_ref[0] + q_ref[1] + y2_ref[...]) * dinv + b2_ref[...]
        col = lax.broadcasted_iota(jnp.int32, (BR, CP), 1)
        logits = jnp.where(col < C, logits, -1e30)
        m = jnp.max(logits, axis=1, keepdims=True)
        s = jnp.sum(jnp.exp(logits - m), axis=1, keepdims=True)
        o_ref[...] = logits - m - jnp.log(s)

    return pl.pallas_call(
        body,
        grid=(GRID,),
        in_specs=[
            pl.BlockSpec((NC, BR, CP), lambda i: (0, i, 0)),
            pl.BlockSpec((NC, BR, CP), lambda i: (0, i, 0)),
            pl.BlockSpec((BR, CP), lambda i: (i, 0)),
            pl.BlockSpec((1, CP), lambda i: (0, 0)),
        ],
        out_specs=pl.BlockSpec((BR, CP), lambda i: (i, 0)),
        out_shape=jax.ShapeDtypeStruct((N, CP), jnp.float32),
    )(q, degp, y2, b2r)


def kernel(x, edge_index, W1, b1, W2, b2):
    src = edge_index[0].astype(jnp.int32)
    dst = edge_index[1].astype(jnp.int32)
    src_p = jnp.pad(src, (0, E_PAD - E)).reshape(NW, NB, BK)
    dst_p = jnp.pad(dst, (0, E_PAD - E), constant_values=TRASH).reshape(NW, NB, BK)
    zeros_f = jnp.zeros((R, F), jnp.float32)
    zeros_c = jnp.zeros((R, CP), jnp.float32)
    ones_rows = jnp.ones((BK, CP), jnp.float32)
    b1r = jnp.reshape(b1, (1, F))
    W2p = jnp.pad(W2, ((0, 0), (0, CP - C)))
    b2r = jnp.reshape(jnp.pad(b2, (0, CP - C)), (1, CP))

    degp = _sc_deg()(dst_p, ones_rows, zeros_c)
    y1 = _tc_scale_in(x, W1, degp)
    p = _sc_prop(F)(y1, src_p, dst_p, zeros_f)
    y2 = _tc_mid(p, degp, y1, b1r, W2p)
    q = _sc_prop(CP)(y2, src_p, dst_p, zeros_c)
    out16 = _tc_out(q, degp, y2, b2r)
    return out16[:, :C]


# speedup vs baseline: 13.4074x; 1.1449x over previous
import functools

import jax
import jax.numpy as jnp
from jax import lax
from jax.experimental import pallas as pl
from jax.experimental.pallas import tpu as pltpu
from jax.experimental.pallas import tpu_sc as plsc

N = 10000
E = 320000
F = 128
C = 10
CP = 16
NC, NS = 2, 16
NW = NC * NS
BK = 128
NB = 80
EPT = NB * BK
E_PAD = NW * EPT
R = 10240
RT = R // NS
TRASH = N + 16
BR = 1000
GRID = N // BR


def _mesh():
    return plsc.VectorSubcoreMesh(
        core_axis_name="c", subcore_axis_name="s", num_cores=NC, num_subcores=NS)


def _sc_deg():
    @functools.partial(
        pl.kernel,
        out_type=jax.ShapeDtypeStruct((NC, R, CP), jnp.float32),
        mesh=_mesh(),
        scratch_types=[
            pltpu.VMEM((NB, BK), jnp.int32),
            pltpu.VMEM((BK, CP), jnp.float32),
            pltpu.SemaphoreType.DMA,
            pltpu.VMEM_SHARED((R, CP), jnp.float32),
        ],
        compiler_params=pltpu.CompilerParams(use_tc_tiling_on_sc=False),
        name="gcn_deg",
    )
    def deg(dst_hbm, ones_hbm, zeros_hbm, degp_hbm, dst_v, ones_v, sem, acc_sh):
        cid = lax.axis_index("c")
        sid = lax.axis_index("s")
        w = cid * NS + sid
        pltpu.sync_copy(dst_hbm.at[w], dst_v)
        pltpu.sync_copy(ones_hbm, ones_v)
        pltpu.sync_copy(zeros_hbm.at[pl.ds(sid * RT, RT)],
                        acc_sh.at[pl.ds(sid * RT, RT)])
        plsc.subcore_barrier()

        K = 8

        def step(i, carry):
            for b in range(K):
                pltpu.async_copy(
                    ones_v, acc_sh.at[dst_v.at[i * K + b]], sem, add=True)
            for b in range(K):
                pltpu.make_async_copy(
                    ones_v, acc_sh.at[dst_v.at[i * K + b]], sem).wait()
            return carry

        lax.fori_loop(0, NB // K, step, 0)
        plsc.subcore_barrier()
        pltpu.sync_copy(acc_sh.at[pl.ds(sid * RT, RT)],
                        degp_hbm.at[cid, pl.ds(sid * RT, RT)])

    return deg


def _sc_prop(D, NBUF, HALVES):
    HB = NB // HALVES
    PD = NBUF // 2

    @functools.partial(
        pl.kernel,
        out_type=jax.ShapeDtypeStruct((NC, R, D), jnp.float32),
        mesh=_mesh(),
        scratch_types=[
            pltpu.VMEM((HB, BK), jnp.int32),
            pltpu.VMEM((HB, BK), jnp.int32),
            pltpu.VMEM((NBUF, BK, D), jnp.float32),
            pltpu.SemaphoreType.DMA((NBUF,)),
            pltpu.SemaphoreType.DMA((NBUF,)),
            pltpu.VMEM_SHARED((R, D), jnp.float32),
        ],
        compiler_params=pltpu.CompilerParams(use_tc_tiling_on_sc=False),
        name=f"gcn_prop_d{D}",
    )
    def prop(y_hbm, src_hbm, dst_hbm, zeros_hbm, p_hbm, src_v, dst_v, buf,
             gsem, ssem, acc_sh):
        cid = lax.axis_index("c")
        sid = lax.axis_index("s")
        w = cid * NS + sid
        pltpu.sync_copy(zeros_hbm.at[pl.ds(sid * RT, RT)],
                        acc_sh.at[pl.ds(sid * RT, RT)])
        plsc.subcore_barrier()

        def gather(j, b):
            pltpu.async_copy(y_hbm.at[src_v.at[j]], buf.at[b], gsem.at[b])

        def gather_wait(j, b):
            pltpu.make_async_copy(
                y_hbm.at[src_v.at[j]], buf.at[b], gsem.at[b]).wait()

        def scatter(j, b):
            pltpu.async_copy(
                buf.at[b], acc_sh.at[dst_v.at[j]], ssem.at[b], add=True)

        def scatter_wait(j, b):
            pltpu.make_async_copy(
                buf.at[b], acc_sh.at[dst_v.at[j]], ssem.at[b]).wait()

        for h in range(HALVES):
            pltpu.sync_copy(src_hbm.at[w, pl.ds(h * HB, HB)], src_v)
            pltpu.sync_copy(dst_hbm.at[w, pl.ds(h * HB, HB)], dst_v)

            for j0 in range(PD):
                gather(j0, j0)

            def step(i, carry):
                for b in range(NBUF):
                    j = NBUF * i + b
                    bw = (b + PD) % NBUF
                    if b < PD:
                        @pl.when(i > 0)
                        def _():
                            scatter_wait(j - PD, bw)
                    else:
                        scatter_wait(j - PD, bw)
                    if b < PD:
                        gather(j + PD, bw)
                    else:
                        @pl.when(i < HB // NBUF - 1)
                        def _():
                            gather(j + PD, bw)
                    gather_wait(j, b)
                    scatter(j, b)
                return carry

            lax.fori_loop(0, HB // NBUF, step, 0)
            for j0 in range(PD):
                scatter_wait(HB - PD + j0, (HB - PD + j0) % NBUF)

        plsc.subcore_barrier()
        pltpu.sync_copy(acc_sh.at[pl.ds(sid * RT, RT)],
                        p_hbm.at[cid, pl.ds(sid * RT, RT)])

    return prop


def _dinv(degp_ref):
    deg = degp_ref[0, :, 0:1] + degp_ref[1, :, 0:1] + 1.0
    return lax.rsqrt(deg)


def _tc_scale_in(x, W1, degp):
    def body(x_ref, w_ref, degp_ref, y_ref):
        dinv = _dinv(degp_ref)
        xw = jnp.dot(x_ref[...], w_ref[...], preferred_element_type=jnp.float32)
        y_ref[...] = xw * dinv

    return pl.pallas_call(
        body,
        grid=(GRID,),
        in_specs=[
            pl.BlockSpec((BR, F), lambda i: (i, 0)),
            pl.BlockSpec((F, F), lambda i: (0, 0)),
            pl.BlockSpec((NC, BR, CP), lambda i: (0, i, 0)),
        ],
        out_specs=pl.BlockSpec((BR, F), lambda i: (i, 0)),
        out_shape=jax.ShapeDtypeStruct((N, F), jnp.float32),
    )(x, W1, degp)


def _tc_mid(p, degp, y1, b1r, W2p):
    def body(p_ref, degp_ref, y1_ref, b1_ref, w2_ref, y2_ref):
        dinv = _dinv(degp_ref)
        acc = p_ref[0] + p_ref[1] + y1_ref[...]
        h = jnp.maximum(acc * dinv + b1_ref[...], 0.0)
        hw = jnp.dot(h, w2_ref[...], preferred_element_type=jnp.float32)
        y2_ref[...] = hw * dinv

    return pl.pallas_call(
        body,
        grid=(GRID,),
        in_specs=[
            pl.BlockSpec((NC, BR, F), lambda i: (0, i, 0)),
            pl.BlockSpec((NC, BR, CP), lambda i: (0, i, 0)),
            pl.BlockSpec((BR, F), lambda i: (i, 0)),
            pl.BlockSpec((1, F), lambda i: (0, 0)),
            pl.BlockSpec((F, CP), lambda i: (0, 0)),
        ],
        out_specs=pl.BlockSpec((BR, CP), lambda i: (i, 0)),
        out_shape=jax.ShapeDtypeStruct((N, CP), jnp.float32),
    )(p, degp, y1, b1r, W2p)


def _tc_out(q, degp, y2, b2r):
    def body(q_ref, degp_ref, y2_ref, b2_ref, o_ref):
        dinv = _dinv(degp_ref)
        logits = (q_ref[0] + q_ref[1] + y2_ref[...]) * dinv + b2_ref[...]
        col = lax.broadcasted_iota(jnp.int32, (BR, CP), 1)
        logits = jnp.where(col < C, logits, -1e30)
        m = jnp.max(logits, axis=1, keepdims=True)
        s = jnp.sum(jnp.exp(logits - m), axis=1, keepdims=True)
        o_ref[...] = logits - m - jnp.log(s)

    return pl.pallas_call(
        body,
        grid=(GRID,),
        in_specs=[
            pl.BlockSpec((NC, BR, CP), lambda i: (0, i, 0)),
            pl.BlockSpec((NC, BR, CP), lambda i: (0, i, 0)),
            pl.BlockSpec((BR, CP), lambda i: (i, 0)),
            pl.BlockSpec((1, CP), lambda i: (0, 0)),
        ],
        out_specs=pl.BlockSpec((BR, CP), lambda i: (i, 0)),
        out_shape=jax.ShapeDtypeStruct((N, CP), jnp.float32),
    )(q, degp, y2, b2r)


def kernel(x, edge_index, W1, b1, W2, b2):
    src = edge_index[0].astype(jnp.int32)
    dst = edge_index[1].astype(jnp.int32)
    src_p = jnp.pad(src, (0, E_PAD - E)).reshape(NW, NB, BK)
    dst_p = jnp.pad(dst, (0, E_PAD - E), constant_values=TRASH).reshape(NW, NB, BK)
    zeros_f = jnp.zeros((R, F), jnp.float32)
    zeros_c = jnp.zeros((R, CP), jnp.float32)
    ones_rows = jnp.ones((BK, CP), jnp.float32)
    b1r = jnp.reshape(b1, (1, F))
    W2p = jnp.pad(W2, ((0, 0), (0, CP - C)))
    b2r = jnp.reshape(jnp.pad(b2, (0, CP - C)), (1, CP))

    degp = _sc_deg()(dst_p, ones_rows, zeros_c)
    y1 = _tc_scale_in(x, W1, degp)
    p = _sc_prop(F, 2, 2)(y1, src_p, dst_p, zeros_f)
    y2 = _tc_mid(p, degp, y1, b1r, W2p)
    q = _sc_prop(CP, 4, 1)(y2, src_p, dst_p, zeros_c)
    out16 = _tc_out(q, degp, y2, b2r)
    return out16[:, :C]


# speedup vs baseline: 30.9446x; 2.3080x over previous
import functools

import jax
import jax.numpy as jnp
from jax import lax
from jax.experimental import pallas as pl
from jax.experimental.pallas import tpu as pltpu
from jax.experimental.pallas import tpu_sc as plsc

N = 10000
E = 320000
F = 128
C = 10
CP = 16
NC, NS = 2, 16
NW = NC * NS
BK = 128
NB = 80
EPT = NB * BK
E_PAD = NW * EPT
R = 10240
RT = R // NS
TRASH = N + 16
BR = 1000
GRID = N // BR
F2 = F // NC


def _mesh():
    return plsc.VectorSubcoreMesh(
        core_axis_name="c", subcore_axis_name="s", num_cores=NC, num_subcores=NS)


def _sc_deg():
    @functools.partial(
        pl.kernel,
        out_type=jax.ShapeDtypeStruct((NC, R, CP), jnp.float32),
        mesh=_mesh(),
        scratch_types=[
            pltpu.VMEM((NB, BK), jnp.int32),
            pltpu.VMEM((BK, CP), jnp.float32),
            pltpu.SemaphoreType.DMA,
            pltpu.VMEM_SHARED((R, CP), jnp.float32),
        ],
        compiler_params=pltpu.CompilerParams(use_tc_tiling_on_sc=False),
        name="gcn_deg",
    )
    def deg(dst_hbm, ones_hbm, zeros_hbm, degp_hbm, dst_v, ones_v, sem, acc_sh):
        cid = lax.axis_index("c")
        sid = lax.axis_index("s")
        w = cid * NS + sid
        pltpu.sync_copy(dst_hbm.at[w], dst_v)
        pltpu.sync_copy(ones_hbm, ones_v)
        pltpu.sync_copy(zeros_hbm.at[pl.ds(sid * RT, RT)],
                        acc_sh.at[pl.ds(sid * RT, RT)])
        plsc.subcore_barrier()

        K = 8

        def step(i, carry):
            for b in range(K):
                pltpu.async_copy(
                    ones_v, acc_sh.at[dst_v.at[i * K + b]], sem, add=True)
            for b in range(K):
                pltpu.make_async_copy(
                    ones_v, acc_sh.at[dst_v.at[i * K + b]], sem).wait()
            return carry

        lax.fori_loop(0, NB // K, step, 0)
        plsc.subcore_barrier()
        pltpu.sync_copy(acc_sh.at[pl.ds(sid * RT, RT)],
                        degp_hbm.at[cid, pl.ds(sid * RT, RT)])

    return deg


def _sc_prop_sp(D, col_split, NBUF, HALVES, name):
    PD = NBUF // 2
    TPB = (E_PAD // NS // BK) if col_split else NB
    HB = TPB // HALVES

    @functools.partial(
        pl.kernel,
        out_type=jax.ShapeDtypeStruct((NC, R, D), jnp.float32),
        mesh=_mesh(),
        scratch_types=[
            pltpu.VMEM((HB, BK), jnp.int32),
            pltpu.VMEM((HB, BK), jnp.int32),
            pltpu.VMEM((NBUF, BK, D), jnp.float32),
            pltpu.SemaphoreType.DMA((NBUF,)),
            pltpu.SemaphoreType.DMA((NBUF,)),
            pltpu.VMEM_SHARED((R, D), jnp.float32),
            pltpu.VMEM_SHARED((R, D), jnp.float32),
        ],
        compiler_params=pltpu.CompilerParams(use_tc_tiling_on_sc=False),
        name=name,
    )
    def prop(y_hbm, src_hbm, dst_hbm, zeros_hbm, p_hbm, src_v, dst_v, buf,
             gsem, ssem, table_sh, acc_sh):
        cid = lax.axis_index("c")
        sid = lax.axis_index("s")
        rows = pl.ds(sid * RT, RT)
        if col_split:
            pltpu.sync_copy(y_hbm.at[cid, rows], table_sh.at[rows])
        else:
            pltpu.sync_copy(y_hbm.at[rows], table_sh.at[rows])
        pltpu.sync_copy(zeros_hbm.at[rows], acc_sh.at[rows])
        plsc.subcore_barrier()

        def gather(j, b):
            pltpu.async_copy(table_sh.at[src_v.at[j]], buf.at[b], gsem.at[b])

        def gather_wait(j, b):
            pltpu.make_async_copy(
                table_sh.at[src_v.at[j]], buf.at[b], gsem.at[b]).wait()

        def scatter(j, b):
            pltpu.async_copy(
                buf.at[b], acc_sh.at[dst_v.at[j]], ssem.at[b], add=True)

        def scatter_wait(j, b):
            pltpu.make_async_copy(
                buf.at[b], acc_sh.at[dst_v.at[j]], ssem.at[b]).wait()

        w = sid if col_split else cid * NS + sid
        for h in range(HALVES):
            pltpu.sync_copy(src_hbm.at[w, pl.ds(h * HB, HB)], src_v)
            pltpu.sync_copy(dst_hbm.at[w, pl.ds(h * HB, HB)], dst_v)

            for j0 in range(PD):
                gather(j0, j0)

            def step(i, carry):
                for b in range(NBUF):
                    j = NBUF * i + b
                    bw = (b + PD) % NBUF
                    if b < PD:
                        @pl.when(i > 0)
                        def _():
                            scatter_wait(j - PD, bw)
                    else:
                        scatter_wait(j - PD, bw)
                    if b < PD:
                        gather(j + PD, bw)
                    else:
                        @pl.when(i < HB // NBUF - 1)
                        def _():
                            gather(j + PD, bw)
                    gather_wait(j, b)
                    scatter(j, b)
                return carry

            lax.fori_loop(0, HB // NBUF, step, 0)
            for j0 in range(PD):
                scatter_wait(HB - PD + j0, (HB - PD + j0) % NBUF)

        plsc.subcore_barrier()
        pltpu.sync_copy(acc_sh.at[rows], p_hbm.at[cid, rows])

    return prop


def _dinv(degp_ref):
    deg = degp_ref[0, :, 0:1] + degp_ref[1, :, 0:1] + 1.0
    return lax.rsqrt(deg)


def _tc_scale_in(x, W1, degp):
    def body(x_ref, w_ref, degp_ref, y_ref):
        dinv = _dinv(degp_ref)
        xw = jnp.dot(x_ref[...], w_ref[...], preferred_element_type=jnp.float32)
        y = xw * dinv
        y_ref[0] = y[:, :F2]
        y_ref[1] = y[:, F2:]

    return pl.pallas_call(
        body,
        grid=(GRID,),
        in_specs=[
            pl.BlockSpec((BR, F), lambda i: (i, 0)),
            pl.BlockSpec((F, F), lambda i: (0, 0)),
            pl.BlockSpec((NC, BR, CP), lambda i: (0, i, 0)),
        ],
        out_specs=pl.BlockSpec((NC, BR, F2), lambda i: (0, i, 0)),
        out_shape=jax.ShapeDtypeStruct((NC, R, F2), jnp.float32),
    )(x, W1, degp)


def _tc_mid(p, degp, y1, b1r, W2p):
    def body(p_ref, degp_ref, y1_ref, b1_ref, w2_ref, y2_ref):
        dinv = _dinv(degp_ref)
        acc = jnp.concatenate(
            [p_ref[0] + y1_ref[0], p_ref[1] + y1_ref[1]], axis=1)
        h = jnp.maximum(acc * dinv + b1_ref[...], 0.0)
        hw = jnp.dot(h, w2_ref[...], preferred_element_type=jnp.float32)
        y2_ref[...] = hw * dinv

    return pl.pallas_call(
        body,
        grid=(GRID,),
        in_specs=[
            pl.BlockSpec((NC, BR, F2), lambda i: (0, i, 0)),
            pl.BlockSpec((NC, BR, CP), lambda i: (0, i, 0)),
            pl.BlockSpec((NC, BR, F2), lambda i: (0, i, 0)),
            pl.BlockSpec((1, F), lambda i: (0, 0)),
            pl.BlockSpec((F, CP), lambda i: (0, 0)),
        ],
        out_specs=pl.BlockSpec((BR, CP), lambda i: (i, 0)),
        out_shape=jax.ShapeDtypeStruct((R, CP), jnp.float32),
    )(p, degp, y1, b1r, W2p)


def _tc_out(q, degp, y2, b2r):
    def body(q_ref, degp_ref, y2_ref, b2_ref, o_ref):
        dinv = _dinv(degp_ref)
        logits = (q_ref[0] + q_ref[1] + y2_ref[...]) * dinv + b2_ref[...]
        col = lax.broadcasted_iota(jnp.int32, (BR, CP), 1)
        logits = jnp.where(col < C, logits, -1e30)
        m = jnp.max(logits, axis=1, keepdims=True)
        s = jnp.sum(jnp.exp(logits - m), axis=1, keepdims=True)
        o_ref[...] = logits - m - jnp.log(s)

    return pl.pallas_call(
        body,
        grid=(GRID,),
        in_specs=[
            pl.BlockSpec((NC, BR, CP), lambda i: (0, i, 0)),
            pl.BlockSpec((NC, BR, CP), lambda i: (0, i, 0)),
            pl.BlockSpec((BR, CP), lambda i: (i, 0)),
            pl.BlockSpec((1, CP), lambda i: (0, 0)),
        ],
        out_specs=pl.BlockSpec((BR, CP), lambda i: (i, 0)),
        out_shape=jax.ShapeDtypeStruct((N, CP), jnp.float32),
    )(q, degp, y2, b2r)


def kernel(x, edge_index, W1, b1, W2, b2):
    src = edge_index[0].astype(jnp.int32)
    dst = edge_index[1].astype(jnp.int32)
    src_p = jnp.pad(src, (0, E_PAD - E)).reshape(NW, NB, BK)
    dst_p = jnp.pad(dst, (0, E_PAD - E), constant_values=TRASH).reshape(NW, NB, BK)
    zeros_h = jnp.zeros((R, F2), jnp.float32)
    zeros_c = jnp.zeros((R, CP), jnp.float32)
    ones_rows = jnp.ones((BK, CP), jnp.float32)
    b1r = jnp.reshape(b1, (1, F))
    W2p = jnp.pad(W2, ((0, 0), (0, CP - C)))
    b2r = jnp.reshape(jnp.pad(b2, (0, CP - C)), (1, CP))

    src16 = src_p.reshape(NS, NB * NC, BK)
    dst16 = dst_p.reshape(NS, NB * NC, BK)

    degp = _sc_deg()(dst_p, ones_rows, zeros_c)
    y1 = _tc_scale_in(x, W1, degp)
    p = _sc_prop_sp(F2, True, 2, 2, "gcn_prop128")(y1, src16, dst16, zeros_h)
    y2 = _tc_mid(p, degp, y1, b1r, W2p)
    q = _sc_prop_sp(CP, False, 2, 1, "gcn_prop16")(y2, src_p, dst_p, zeros_c)
    out16 = _tc_out(q, degp, y2, b2r)
    return out16[:, :C]
